# baseline (speedup 1.0000x reference)
import functools

import jax
import jax.numpy as jnp
from jax import lax
from jax.experimental import pallas as pl
from jax.experimental.pallas import tpu as pltpu
from jax.experimental.pallas import tpu_sc as plsc

NC = 2
NS = 16
NW = NC * NS
CH = 1024
ROWS = CH // 128


def _scatter_body(npad, totb, ei_hbm, x_hbm, zero_hbm, agg_hbm,
                  sidx, didx, rv, sha, gsem, ssem):
    cid = lax.axis_index("c")
    sid = lax.axis_index("s")
    wid = cid * NS + sid
    sp = npad // NS
    nb = sid * sp
    pltpu.sync_copy(zero_hbm.at[pl.ds(nb, sp)], sha.at[pl.ds(nb, sp)])
    plsc.subcore_barrier()

    wpb = -(-totb // NW)
    b0 = wid * wpb
    nbat = jnp.maximum(jnp.minimum(wpb, totb - b0), 0)
    nfull = nbat // ROWS

    def step(i, carry):
        r0 = b0 + i * ROWS
        pltpu.sync_copy(ei_hbm.at[0, pl.ds(r0, ROWS)], sidx)
        pltpu.sync_copy(ei_hbm.at[1, pl.ds(r0, ROWS)], didx)
        gds = [pltpu.async_copy(x_hbm.at[sidx.at[j]],
                                rv.at[pl.ds(j * 128, 128)], gsem)
               for j in range(ROWS)]
        for d in gds:
            d.wait()
        sds = [pltpu.async_copy(rv.at[pl.ds(j * 128, 128)],
                                sha.at[didx.at[j]], ssem, add=True)
               for j in range(ROWS)]
        for d in sds:
            d.wait()
        return carry

    lax.fori_loop(0, nfull, step, 0)

    def tail(k, carry):
        r0 = b0 + k
        pltpu.sync_copy(ei_hbm.at[0, pl.ds(r0, 1)], sidx.at[pl.ds(0, 1)])
        pltpu.sync_copy(ei_hbm.at[1, pl.ds(r0, 1)], didx.at[pl.ds(0, 1)])
        pltpu.async_copy(x_hbm.at[sidx.at[0]],
                         rv.at[pl.ds(0, 128)], gsem).wait()
        pltpu.async_copy(rv.at[pl.ds(0, 128)],
                         sha.at[didx.at[0]], ssem, add=True).wait()
        return carry

    lax.fori_loop(nfull * ROWS, nbat, tail, 0)
    plsc.subcore_barrier()
    pltpu.sync_copy(sha.at[pl.ds(nb, sp)], agg_hbm.at[cid, pl.ds(nb, sp)])


def _make_scatter(npad, e):
    totb = e // 128
    mesh = plsc.VectorSubcoreMesh(core_axis_name="c", subcore_axis_name="s",
                                  num_cores=NC, num_subcores=NS)
    return pl.kernel(
        functools.partial(_scatter_body, npad, totb),
        out_type=jax.ShapeDtypeStruct((NC, npad, 8), jnp.float32),
        mesh=mesh,
        compiler_params=pltpu.CompilerParams(use_tc_tiling_on_sc=False),
        scratch_types=[
            pltpu.VMEM((ROWS, 128), jnp.int32),
            pltpu.VMEM((ROWS, 128), jnp.int32),
            pltpu.VMEM((CH, 8), jnp.float32),
            pltpu.VMEM_SHARED((npad, 8), jnp.float32),
            pltpu.SemaphoreType.DMA,
            pltpu.SemaphoreType.DMA,
        ],
    )


def _dense_a_body(x_ref, agg_ref, w_ref, m_ref, o_ref):
    ag = agg_ref[0] + agg_ref[1]
    o_ref[...] = jnp.maximum(
        x_ref[...] @ w_ref[...] + ag @ m_ref[...], 0.0)


def _dense_b_body(h_ref, agg_ref, u_ref, v_ref, o_ref):
    ag = agg_ref[0] + agg_ref[1]
    t = jnp.maximum(h_ref[...] @ u_ref[...] + ag @ v_ref[...], 0.0)
    t3 = t[:, :3]
    m = jnp.max(t3, axis=1, keepdims=True)
    e = jnp.exp(t3 - m)
    s = jnp.sum(e, axis=1, keepdims=True)
    o_ref[...] = jnp.concatenate(
        [e / s, jnp.zeros_like(t[:, :5])], axis=1)


def _dense_call(body, npad, xp, agg, a4, b4):
    br = 1024
    grid = npad // br
    return pl.pallas_call(
        body,
        grid=(grid,),
        in_specs=[
            pl.BlockSpec((br, 8), lambda i: (i, 0)),
            pl.BlockSpec((NC, br, 8), lambda i: (0, i, 0)),
            pl.BlockSpec((8, 8), lambda i: (0, 0)),
            pl.BlockSpec((8, 8), lambda i: (0, 0)),
        ],
        out_specs=pl.BlockSpec((br, 8), lambda i: (i, 0)),
        out_shape=jax.ShapeDtypeStruct((npad, 8), jnp.float32),
    )(xp, agg, a4, b4)


def _pairs_body(pw, g_hbm, p0_hbm, p1_hbm, wb_hbm, o_hbm,
                p0v, p1v, idx6, col6, wbv, o0v, o1v, sem):
    cid = lax.axis_index("c")
    sid = lax.axis_index("s")
    wid = cid * NS + sid
    base = wid * pw
    pltpu.sync_copy(p0_hbm.at[pl.ds(base, pw)], p0v)
    pltpu.sync_copy(p1_hbm.at[pl.ds(base, pw)], p1v)
    pltpu.sync_copy(wb_hbm, wbv)
    for s, pv in enumerate((p0v, p1v)):
        for c in range(3):
            for i in range(pw // 16):
                idx6[s * 3 + c][pl.ds(i * 16, 16)] = \
                    pv[pl.ds(i * 16, 16)] * 8 + c
    ds_ = [pltpu.async_copy(g_hbm.at[idx6[k]], col6[k], sem)
           for k in range(6)]
    for d in ds_:
        d.wait()
    for i in range(pw // 16):
        ac = [col6[c][pl.ds(i * 16, 16)] for c in range(3)]
        bc = [col6[3 + c][pl.ds(i * 16, 16)] for c in range(3)]
        z = [ac[c] * bc[c] for c in range(3)] + \
            [ac[c] + bc[c] for c in range(3)]
        v = [jnp.maximum(
            sum(z[j] * wbv[c * 6 + j] for j in range(6)) + wbv[18 + c], 0.0)
            for c in range(3)]
        o = [sum(v[c] * wbv[21 + k * 3 + c] for c in range(3)) + wbv[27 + k]
             for k in range(2)]
        m = jnp.maximum(o[0], o[1])
        e0 = jnp.exp(o[0] - m)
        e1 = jnp.exp(o[1] - m)
        s = e0 + e1
        o0v[pl.ds(i * 16, 16)] = e0 / s
        o1v[pl.ds(i * 16, 16)] = e1 / s
    pltpu.sync_copy(o0v, o_hbm.at[0, pl.ds(base, pw)])
    pltpu.sync_copy(o1v, o_hbm.at[1, pl.ds(base, pw)])


def _make_pairs(npad, b):
    pw = b // NW
    mesh = plsc.VectorSubcoreMesh(core_axis_name="c", subcore_axis_name="s",
                                  num_cores=NC, num_subcores=NS)
    return pl.kernel(
        functools.partial(_pairs_body, pw),
        out_type=jax.ShapeDtypeStruct((2, b), jnp.float32),
        mesh=mesh,
        compiler_params=pltpu.CompilerParams(use_tc_tiling_on_sc=False),
        scratch_types=[
            pltpu.VMEM((pw,), jnp.int32),
            pltpu.VMEM((pw,), jnp.int32),
            [pltpu.VMEM((pw,), jnp.int32) for _ in range(6)],
            [pltpu.VMEM((pw,), jnp.float32) for _ in range(6)],
            pltpu.VMEM((32, 16), jnp.float32),
            pltpu.VMEM((pw,), jnp.float32),
            pltpu.VMEM((pw,), jnp.float32),
            pltpu.SemaphoreType.DMA,
        ],
    )


def kernel(x, edge_index, pairs, W, M, U, V, W1, b1, W2, b2):
    n = x.shape[0]
    e = edge_index.shape[1]
    b = pairs.shape[0]
    npad = ((n + 2048) // 2048) * 2048

    ei = edge_index.astype(jnp.int32).reshape(2, e // 128, 128)
    prs = pairs.astype(jnp.int32)
    p0 = prs[:, 0]
    p1 = prs[:, 1]

    xp = jnp.pad(x, ((0, npad - n), (0, 5)))
    zrow = jnp.zeros((npad, 8), jnp.float32)
    w4 = jnp.pad(W, ((0, 5), (0, 5)))
    m4 = jnp.pad(M, ((0, 5), (0, 5)))
    u4 = jnp.pad(U, ((0, 5), (0, 5)))
    v4 = jnp.pad(V, ((0, 5), (0, 5)))
    w_all = jnp.concatenate(
        [W1.reshape(-1), b1, W2.reshape(-1), b2,
         jnp.zeros((3,), jnp.float32)])
    wb = jnp.broadcast_to(w_all[:, None], (32, 16))

    scatter = _make_scatter(npad, e)
    agg1 = scatter(ei, xp, zrow)
    hp = _dense_call(_dense_a_body, npad, xp, agg1, w4, m4)
    agg2 = scatter(ei, hp, zrow)
    gp = _dense_call(_dense_b_body, npad, hp, agg2, u4, v4)
    o = _make_pairs(npad, b)(gp.reshape(-1), p0, p1, wb)
    return o.T


# speedup vs baseline: 1.4219x; 1.4219x over previous
import functools

import jax
import jax.numpy as jnp
from jax import lax
from jax.experimental import pallas as pl
from jax.experimental.pallas import tpu as pltpu
from jax.experimental.pallas import tpu_sc as plsc

NC = 2
NS = 16
NW = NC * NS
CH = 1024
ROWS = CH // 128


def _scatter_body(npad, totb, ei_hbm, x_hbm, zero_hbm, agg_hbm,
                  sidx, didx, rv, shx, sha, gsem, ssem):
    cid = lax.axis_index("c")
    sid = lax.axis_index("s")
    wid = cid * NS + sid
    sp = npad // NS
    nb = sid * sp
    pltpu.sync_copy(x_hbm.at[pl.ds(nb, sp)], shx.at[pl.ds(nb, sp)])
    pltpu.sync_copy(zero_hbm.at[pl.ds(nb, sp)], sha.at[pl.ds(nb, sp)])
    plsc.subcore_barrier()

    wpb = -(-totb // NW)
    b0 = wid * wpb
    nbat = jnp.maximum(jnp.minimum(wpb, totb - b0), 0)
    nfull = nbat // ROWS

    def load_and_gather(i, slot):
        r0 = b0 + i * ROWS
        so = slot * ROWS
        pltpu.sync_copy(ei_hbm.at[0, pl.ds(r0, ROWS)],
                        sidx.at[pl.ds(so, ROWS)])
        pltpu.sync_copy(ei_hbm.at[1, pl.ds(r0, ROWS)],
                        didx.at[pl.ds(so, ROWS)])
        return [pltpu.async_copy(shx.at[sidx.at[so + j]],
                                 rv.at[pl.ds((so + j) * 128, 128)], gsem)
                for j in range(ROWS)]

    def scatter(slot):
        so = slot * ROWS
        return [pltpu.async_copy(rv.at[pl.ds((so + j) * 128, 128)],
                                 sha.at[didx.at[so + j]], ssem, add=True)
                for j in range(ROWS)]

    def step2(i, carry):
        for slot in range(2):
            ii = i * 2 + slot
            gds = load_and_gather(ii, slot)
            for d in gds:
                d.wait()
            sds = scatter(slot)
            for d in sds:
                d.wait()
        return carry

    npairs = nfull // 2

    def pipe(i, carry):
        ii = i * 2
        g0 = load_and_gather(ii, 0)
        for d in g0:
            d.wait()
        s0 = scatter(0)
        g1 = load_and_gather(ii + 1, 1)
        for d in s0:
            d.wait()
        for d in g1:
            d.wait()
        s1 = scatter(1)
        for d in s1:
            d.wait()
        return carry

    lax.fori_loop(0, npairs, pipe, 0)

    def step(i, carry):
        gds = load_and_gather(i, 0)
        for d in gds:
            d.wait()
        sds = scatter(0)
        for d in sds:
            d.wait()
        return carry

    lax.fori_loop(npairs * 2, nfull, step, 0)

    def tail(k, carry):
        r0 = b0 + k
        pltpu.sync_copy(ei_hbm.at[0, pl.ds(r0, 1)], sidx.at[pl.ds(0, 1)])
        pltpu.sync_copy(ei_hbm.at[1, pl.ds(r0, 1)], didx.at[pl.ds(0, 1)])
        pltpu.async_copy(shx.at[sidx.at[0]],
                         rv.at[pl.ds(0, 128)], gsem).wait()
        pltpu.async_copy(rv.at[pl.ds(0, 128)],
                         sha.at[didx.at[0]], ssem, add=True).wait()
        return carry

    lax.fori_loop(nfull * ROWS, nbat, tail, 0)
    plsc.subcore_barrier()
    pltpu.sync_copy(sha.at[pl.ds(nb, sp)], agg_hbm.at[cid, pl.ds(nb, sp)])


def _make_scatter(npad, e):
    totb = e // 128
    mesh = plsc.VectorSubcoreMesh(core_axis_name="c", subcore_axis_name="s",
                                  num_cores=NC, num_subcores=NS)
    return pl.kernel(
        functools.partial(_scatter_body, npad, totb),
        out_type=jax.ShapeDtypeStruct((NC, npad, 8), jnp.float32),
        mesh=mesh,
        compiler_params=pltpu.CompilerParams(use_tc_tiling_on_sc=False),
        scratch_types=[
            pltpu.VMEM((2 * ROWS, 128), jnp.int32),
            pltpu.VMEM((2 * ROWS, 128), jnp.int32),
            pltpu.VMEM((2 * CH, 8), jnp.float32),
            pltpu.VMEM_SHARED((npad, 8), jnp.float32),
            pltpu.VMEM_SHARED((npad, 8), jnp.float32),
            pltpu.SemaphoreType.DMA,
            pltpu.SemaphoreType.DMA,
        ],
    )


def _dense_a_body(x_ref, agg_ref, w_ref, m_ref, o_ref):
    ag = agg_ref[0] + agg_ref[1]
    o_ref[...] = jnp.maximum(
        x_ref[...] @ w_ref[...] + ag @ m_ref[...], 0.0)


def _dense_b_body(h_ref, agg_ref, u_ref, v_ref, o_ref):
    ag = agg_ref[0] + agg_ref[1]
    t = jnp.maximum(h_ref[...] @ u_ref[...] + ag @ v_ref[...], 0.0)
    t3 = t[:, :3]
    m = jnp.max(t3, axis=1, keepdims=True)
    e = jnp.exp(t3 - m)
    s = jnp.sum(e, axis=1, keepdims=True)
    o_ref[...] = jnp.concatenate(
        [e / s, jnp.zeros_like(t[:, :5])], axis=1)


def _dense_call(body, npad, xp, agg, a4, b4):
    br = 1024
    grid = npad // br
    return pl.pallas_call(
        body,
        grid=(grid,),
        in_specs=[
            pl.BlockSpec((br, 8), lambda i: (i, 0)),
            pl.BlockSpec((NC, br, 8), lambda i: (0, i, 0)),
            pl.BlockSpec((8, 8), lambda i: (0, 0)),
            pl.BlockSpec((8, 8), lambda i: (0, 0)),
        ],
        out_specs=pl.BlockSpec((br, 8), lambda i: (i, 0)),
        out_shape=jax.ShapeDtypeStruct((npad, 8), jnp.float32),
    )(xp, agg, a4, b4)


def _pairs_body(pw, g_hbm, p0_hbm, p1_hbm, wb_hbm, o_hbm,
                p0v, p1v, idx6, col6, wbv, o0v, o1v, sem):
    cid = lax.axis_index("c")
    sid = lax.axis_index("s")
    wid = cid * NS + sid
    base = wid * pw
    pltpu.sync_copy(p0_hbm.at[pl.ds(base, pw)], p0v)
    pltpu.sync_copy(p1_hbm.at[pl.ds(base, pw)], p1v)
    pltpu.sync_copy(wb_hbm, wbv)
    for s, pv in enumerate((p0v, p1v)):
        for c in range(3):
            for i in range(pw // 16):
                idx6[s * 3 + c][pl.ds(i * 16, 16)] = \
                    pv[pl.ds(i * 16, 16)] * 8 + c
    ds_ = [pltpu.async_copy(g_hbm.at[idx6[k]], col6[k], sem)
           for k in range(6)]
    for d in ds_:
        d.wait()
    for i in range(pw // 16):
        ac = [col6[c][pl.ds(i * 16, 16)] for c in range(3)]
        bc = [col6[3 + c][pl.ds(i * 16, 16)] for c in range(3)]
        z = [ac[c] * bc[c] for c in range(3)] + \
            [ac[c] + bc[c] for c in range(3)]
        v = [jnp.maximum(
            sum(z[j] * wbv[c * 6 + j] for j in range(6)) + wbv[18 + c], 0.0)
            for c in range(3)]
        o = [sum(v[c] * wbv[21 + k * 3 + c] for c in range(3)) + wbv[27 + k]
             for k in range(2)]
        m = jnp.maximum(o[0], o[1])
        e0 = jnp.exp(o[0] - m)
        e1 = jnp.exp(o[1] - m)
        s = e0 + e1
        o0v[pl.ds(i * 16, 16)] = e0 / s
        o1v[pl.ds(i * 16, 16)] = e1 / s
    pltpu.sync_copy(o0v, o_hbm.at[0, pl.ds(base, pw)])
    pltpu.sync_copy(o1v, o_hbm.at[1, pl.ds(base, pw)])


def _make_pairs(npad, b):
    pw = b // NW
    mesh = plsc.VectorSubcoreMesh(core_axis_name="c", subcore_axis_name="s",
                                  num_cores=NC, num_subcores=NS)
    return pl.kernel(
        functools.partial(_pairs_body, pw),
        out_type=jax.ShapeDtypeStruct((2, b), jnp.float32),
        mesh=mesh,
        compiler_params=pltpu.CompilerParams(use_tc_tiling_on_sc=False),
        scratch_types=[
            pltpu.VMEM((pw,), jnp.int32),
            pltpu.VMEM((pw,), jnp.int32),
            [pltpu.VMEM((pw,), jnp.int32) for _ in range(6)],
            [pltpu.VMEM((pw,), jnp.float32) for _ in range(6)],
            pltpu.VMEM((32, 16), jnp.float32),
            pltpu.VMEM((pw,), jnp.float32),
            pltpu.VMEM((pw,), jnp.float32),
            pltpu.SemaphoreType.DMA,
        ],
    )


def kernel(x, edge_index, pairs, W, M, U, V, W1, b1, W2, b2):
    n = x.shape[0]
    e = edge_index.shape[1]
    b = pairs.shape[0]
    npad = ((n + 2048) // 2048) * 2048

    ei = edge_index.astype(jnp.int32).reshape(2, e // 128, 128)
    prs = pairs.astype(jnp.int32)
    p0 = prs[:, 0]
    p1 = prs[:, 1]

    xp = jnp.pad(x, ((0, npad - n), (0, 5)))
    zrow = jnp.zeros((npad, 8), jnp.float32)
    w4 = jnp.pad(W, ((0, 5), (0, 5)))
    m4 = jnp.pad(M, ((0, 5), (0, 5)))
    u4 = jnp.pad(U, ((0, 5), (0, 5)))
    v4 = jnp.pad(V, ((0, 5), (0, 5)))
    w_all = jnp.concatenate(
        [W1.reshape(-1), b1, W2.reshape(-1), b2,
         jnp.zeros((3,), jnp.float32)])
    wb = jnp.broadcast_to(w_all[:, None], (32, 16))

    scatter = _make_scatter(npad, e)
    agg1 = scatter(ei, xp, zrow)
    hp = _dense_call(_dense_a_body, npad, xp, agg1, w4, m4)
    agg2 = scatter(ei, hp, zrow)
    gp = _dense_call(_dense_b_body, npad, hp, agg2, u4, v4)
    o = _make_pairs(npad, b)(gp.reshape(-1), p0, p1, wb)
    return o.T
